# Initial kernel scaffold; baseline (speedup 1.0000x reference)
import jax
import jax.numpy as jnp
from jax.experimental import pallas as pl


def kernel(selfies, properties, values, mask, E_tok, E_prop, w_val, W1, b1, W2, b2):
    raise NotImplementedError("write your pallas kernel here")


# speedup vs baseline: 5.9708x; 5.9708x over previous
import functools

import jax
import jax.numpy as jnp
from jax import lax
from jax.experimental import pallas as pl
from jax.experimental.pallas import tpu as pltpu
from jax.experimental.pallas import tpu_sc as plsc

NPROP = 16
VOCAB = 512
HDIM = 2048
HID = 1024

BJ = 512
TBLK = 256

SC_CORES = 2
SC_SUBCORES = 16
SC_WORKERS = SC_CORES * SC_SUBCORES
SC_CHUNK = 32


def _table_body(stacked_ref, eprop_ref, w1_ref, a_ref, u_ref, ep_ref):
    s = stacked_ref[...].astype(jnp.bfloat16)
    w = w1_ref[0].astype(jnp.bfloat16)
    m = lax.dot_general(s, w, (((1,), (0,)), ((), ())),
                        preferred_element_type=jnp.float32)
    a_ref[0] = m[:VOCAB]
    u_ref[0, 0] = m[VOCAB]
    ep = lax.dot_general(eprop_ref[0].astype(jnp.bfloat16), w,
                         (((1,), (0,)), ((), ())),
                         preferred_element_type=jnp.float32)
    ep_ref[0] = ep


def _build_tables(E_tok, E_prop, w_val, W1):
    stacked = jnp.concatenate([E_tok, w_val[None, :]], axis=0)
    eprop3 = E_prop[:, None, :]
    return pl.pallas_call(
        _table_body,
        grid=(NPROP, HID // BJ),
        in_specs=[
            pl.BlockSpec((VOCAB + 1, HDIM), lambda e, j: (0, 0)),
            pl.BlockSpec((1, 1, HDIM), lambda e, j: (e, 0, 0)),
            pl.BlockSpec((1, HDIM, BJ), lambda e, j: (e, 0, j)),
        ],
        out_specs=[
            pl.BlockSpec((1, VOCAB, BJ), lambda e, j: (e, 0, j)),
            pl.BlockSpec((1, 1, BJ), lambda e, j: (e, 0, j)),
            pl.BlockSpec((1, 1, BJ), lambda e, j: (e, 0, j)),
        ],
        out_shape=[
            jax.ShapeDtypeStruct((NPROP, VOCAB, HID), jnp.float32),
            jax.ShapeDtypeStruct((NPROP, 1, HID), jnp.float32),
            jax.ShapeDtypeStruct((NPROP, 1, HID), jnp.float32),
        ],
    )(stacked, eprop3, W1)


def _sc_gather(table, props_flat, selfies_flat):
    ntok = props_flat.shape[0]
    per_w = ntok // SC_WORKERS
    n_chunks = per_w // SC_CHUNK
    mesh = plsc.VectorSubcoreMesh(core_axis_name="c", subcore_axis_name="s")

    @functools.partial(
        pl.kernel, mesh=mesh,
        out_type=jax.ShapeDtypeStruct((ntok, HID), jnp.float32),
        scratch_types=[
            pltpu.VMEM((SC_CHUNK,), jnp.int32),
            pltpu.VMEM((SC_CHUNK,), jnp.int32),
            pltpu.VMEM((SC_CHUNK,), jnp.int32),
            pltpu.VMEM((SC_CHUNK, HID), jnp.float32),
            pltpu.SemaphoreType.DMA,
        ],
    )
    def k(table_hbm, p_hbm, s_hbm, out_hbm, p_v, s_v, idx_v, rows_v, sem):
        wid = lax.axis_index("s") * SC_CORES + lax.axis_index("c")
        base = wid * per_w

        @pl.loop(0, n_chunks)
        def _chunk(c):
            b = base + c * SC_CHUNK
            pltpu.sync_copy(p_hbm.at[pl.ds(b, SC_CHUNK)], p_v)
            pltpu.sync_copy(s_hbm.at[pl.ds(b, SC_CHUNK)], s_v)

            @pl.loop(0, SC_CHUNK, step=16)
            def _lane(j):
                sl = pl.ds(j, 16)
                idx_v.at[sl][...] = p_v.at[sl][...] * VOCAB + s_v.at[sl][...]

            pltpu.async_copy(table_hbm.at[idx_v], rows_v, sem).wait()
            pltpu.sync_copy(rows_v, out_hbm.at[pl.ds(b, SC_CHUNK)])

    return k(table, props_flat, selfies_flat)


def _head_body(g_ref, p_ref, v_ref, m_ref, tab_ref, o_ref):
    props = p_ref[...]
    iota = lax.broadcasted_iota(jnp.int32, (TBLK, NPROP), 1)
    oh = (iota == props).astype(jnp.bfloat16)
    tab = tab_ref[...].astype(jnp.bfloat16)
    tbl = lax.dot_general(oh, tab, (((1,), (0,)), ((), ())),
                          preferred_element_type=jnp.float32)
    u = tbl[:, :HID]
    epr = tbl[:, HID:2 * HID]
    b1g = tbl[:, 2 * HID:3 * HID]
    vv = tbl[:, 3 * HID:4 * HID]
    b2g = tbl[:, 4 * HID:4 * HID + 1]
    pre = m_ref[...] * (g_ref[...] + epr + v_ref[...] * u) + b1g
    h = jnp.maximum(pre, 0.0)
    o_ref[...] = jnp.sum(h * vv, axis=1, keepdims=True) + b2g


def _head(G, pf, vf, mf, tab):
    ntok = G.shape[0]
    tw = tab.shape[1]
    return pl.pallas_call(
        _head_body,
        grid=(ntok // TBLK,),
        in_specs=[
            pl.BlockSpec((TBLK, HID), lambda i: (i, 0)),
            pl.BlockSpec((TBLK, 1), lambda i: (i, 0)),
            pl.BlockSpec((TBLK, 1), lambda i: (i, 0)),
            pl.BlockSpec((TBLK, 1), lambda i: (i, 0)),
            pl.BlockSpec((NPROP, tw), lambda i: (0, 0)),
        ],
        out_specs=pl.BlockSpec((TBLK, 1), lambda i: (i, 0)),
        out_shape=jax.ShapeDtypeStruct((ntok, 1), jnp.float32),
    )(G, pf, vf, mf, tab)


def kernel(selfies, properties, values, mask, E_tok, E_prop, w_val, W1, b1, W2, b2):
    B, S = selfies.shape
    ntok = B * S
    sf = selfies.reshape(ntok).astype(jnp.int32)
    pf = properties.reshape(ntok).astype(jnp.int32)
    vf = values.reshape(ntok, 1)
    mf = mask.reshape(ntok, 1).astype(jnp.float32)

    A, U3, EP3 = _build_tables(E_tok, E_prop, w_val, W1)
    table = A.reshape(NPROP * VOCAB, HID)
    G = _sc_gather(table, pf, sf)

    tab = jnp.concatenate(
        [U3[:, 0], EP3[:, 0], b1, W2[:, :, 0],
         jnp.pad(b2, ((0, 0), (0, 127)))], axis=1)
    out = _head(G, pf[:, None], vf, mf, tab)
    return out.reshape(B, S, 1)


# speedup vs baseline: 6.8259x; 1.1432x over previous
import functools

import jax
import jax.numpy as jnp
from jax import lax
from jax.experimental import pallas as pl
from jax.experimental.pallas import tpu as pltpu
from jax.experimental.pallas import tpu_sc as plsc

NPROP = 16
VOCAB = 512
HDIM = 2048
HID = 1024

BJ = 512
TBLK = 512

SC_CORES = 2
SC_SUBCORES = 16
SC_WORKERS = SC_CORES * SC_SUBCORES
SC_CHUNK = 32


def _table_body(stacked_ref, eprop_ref, w1_ref, a_ref, u_ref, ep_ref):
    s = stacked_ref[...]
    w = w1_ref[0].astype(jnp.bfloat16)
    m = lax.dot_general(s, w, (((1,), (0,)), ((), ())),
                        preferred_element_type=jnp.float32)
    a_ref[0] = m[:VOCAB]
    u_ref[0, 0] = m[VOCAB]
    ep = lax.dot_general(eprop_ref[0], w,
                         (((1,), (0,)), ((), ())),
                         preferred_element_type=jnp.float32)
    ep_ref[0] = ep


def _build_tables(E_tok, E_prop, w_val, W1):
    stacked = jnp.concatenate(
        [E_tok, w_val[None, :]], axis=0).astype(jnp.bfloat16)
    eprop3 = E_prop[:, None, :].astype(jnp.bfloat16)
    return pl.pallas_call(
        _table_body,
        grid=(NPROP, HID // BJ),
        in_specs=[
            pl.BlockSpec((VOCAB + 1, HDIM), lambda e, j: (0, 0)),
            pl.BlockSpec((1, 1, HDIM), lambda e, j: (e, 0, 0)),
            pl.BlockSpec((1, HDIM, BJ), lambda e, j: (e, 0, j)),
        ],
        out_specs=[
            pl.BlockSpec((1, VOCAB, BJ), lambda e, j: (e, 0, j)),
            pl.BlockSpec((1, 1, BJ), lambda e, j: (e, 0, j)),
            pl.BlockSpec((1, 1, BJ), lambda e, j: (e, 0, j)),
        ],
        out_shape=[
            jax.ShapeDtypeStruct((NPROP, VOCAB, HID), jnp.float32),
            jax.ShapeDtypeStruct((NPROP, 1, HID), jnp.float32),
            jax.ShapeDtypeStruct((NPROP, 1, HID), jnp.float32),
        ],
    )(stacked, eprop3, W1)


def _sc_gather(table, props_flat, selfies_flat):
    ntok = props_flat.shape[0]
    per_w = ntok // SC_WORKERS
    n_chunks = per_w // SC_CHUNK
    mesh = plsc.VectorSubcoreMesh(core_axis_name="c", subcore_axis_name="s")

    @functools.partial(
        pl.kernel, mesh=mesh,
        out_type=jax.ShapeDtypeStruct((ntok, HID), jnp.float32),
        scratch_types=[
            pltpu.VMEM((SC_CHUNK,), jnp.int32),
            pltpu.VMEM((SC_CHUNK,), jnp.int32),
            pltpu.VMEM((SC_CHUNK,), jnp.int32),
            pltpu.VMEM((SC_CHUNK,), jnp.int32),
            pltpu.VMEM((SC_CHUNK, HID), jnp.float32),
            pltpu.VMEM((SC_CHUNK, HID), jnp.float32),
            pltpu.SemaphoreType.DMA,
            pltpu.SemaphoreType.DMA,
        ],
    )
    def k(table_hbm, p_hbm, s_hbm, out_hbm,
          p_v, s_v, idx0, idx1, rows0, rows1, sem0, sem1):
        wid = lax.axis_index("s") * SC_CORES + lax.axis_index("c")
        base = wid * per_w
        bufs = ((idx0, rows0, sem0), (idx1, rows1, sem1))

        def _prep_and_fire(cc, idx_v, rows_v, sem):
            b = base + cc * SC_CHUNK
            pltpu.sync_copy(p_hbm.at[pl.ds(b, SC_CHUNK)], p_v)
            pltpu.sync_copy(s_hbm.at[pl.ds(b, SC_CHUNK)], s_v)

            @pl.loop(0, SC_CHUNK, step=16)
            def _lane(j):
                sl = pl.ds(j, 16)
                idx_v.at[sl][...] = p_v.at[sl][...] * VOCAB + s_v.at[sl][...]

            pltpu.async_copy(table_hbm.at[idx_v], rows_v, sem)

        _prep_and_fire(0, idx0, rows0, sem0)
        _prep_and_fire(1, idx1, rows1, sem1)

        @pl.loop(0, n_chunks, step=2)
        def _chunk(c):
            for bi in range(2):
                idx_v, rows_v, sem = bufs[bi]
                cc = c + bi
                pltpu.make_async_copy(
                    table_hbm.at[idx_v], rows_v, sem).wait()
                pltpu.sync_copy(
                    rows_v, out_hbm.at[pl.ds(base + cc * SC_CHUNK, SC_CHUNK)])

                @pl.when(cc + 2 < n_chunks)
                def _refill():
                    _prep_and_fire(cc + 2, idx_v, rows_v, sem)

    return k(table, props_flat, selfies_flat)


def _head_body(g_ref, p_ref, v_ref, m_ref, tab_ref, o_ref):
    props = p_ref[...]
    iota = lax.broadcasted_iota(jnp.int32, (TBLK, NPROP), 1)
    oh = (iota == props).astype(jnp.bfloat16)
    tab = tab_ref[...].astype(jnp.bfloat16)
    tbl = lax.dot_general(oh, tab, (((1,), (0,)), ((), ())),
                          preferred_element_type=jnp.float32)
    u = tbl[:, :HID]
    epr = tbl[:, HID:2 * HID]
    b1g = tbl[:, 2 * HID:3 * HID]
    vv = tbl[:, 3 * HID:4 * HID]
    b2g = tbl[:, 4 * HID:4 * HID + 1]
    pre = m_ref[...] * (g_ref[...] + epr + v_ref[...] * u) + b1g
    h = jnp.maximum(pre, 0.0)
    o_ref[...] = jnp.sum(h * vv, axis=1, keepdims=True) + b2g


def _head(G, pf, vf, mf, tab):
    ntok = G.shape[0]
    tw = tab.shape[1]
    return pl.pallas_call(
        _head_body,
        grid=(ntok // TBLK,),
        in_specs=[
            pl.BlockSpec((TBLK, HID), lambda i: (i, 0)),
            pl.BlockSpec((TBLK, 1), lambda i: (i, 0)),
            pl.BlockSpec((TBLK, 1), lambda i: (i, 0)),
            pl.BlockSpec((TBLK, 1), lambda i: (i, 0)),
            pl.BlockSpec((NPROP, tw), lambda i: (0, 0)),
        ],
        out_specs=pl.BlockSpec((TBLK, 1), lambda i: (i, 0)),
        out_shape=jax.ShapeDtypeStruct((ntok, 1), jnp.float32),
    )(G, pf, vf, mf, tab)


def kernel(selfies, properties, values, mask, E_tok, E_prop, w_val, W1, b1, W2, b2):
    B, S = selfies.shape
    ntok = B * S
    sf = selfies.reshape(ntok).astype(jnp.int32)
    pf = properties.reshape(ntok).astype(jnp.int32)
    vf = values.reshape(ntok, 1)
    mf = mask.reshape(ntok, 1).astype(jnp.float32)

    A, U3, EP3 = _build_tables(E_tok, E_prop, w_val, W1)
    table = A.reshape(NPROP * VOCAB, HID)
    G = _sc_gather(table, pf, sf)

    tab = jnp.concatenate(
        [U3[:, 0], EP3[:, 0], b1, W2[:, :, 0],
         jnp.pad(b2, ((0, 0), (0, 127)))], axis=1)
    out = _head(G, pf[:, None], vf, mf, tab)
    return out.reshape(B, S, 1)
